# baseline (speedup 1.0000x reference)
import functools

import jax
import jax.numpy as jnp
from jax import lax
from jax.experimental import pallas as pl
from jax.experimental.pallas import tpu as pltpu
from jax.experimental.pallas import tpu_sc as plsc

N_PERTS = 1000000
N_LATENT = 64
B = 16384
M = 2
PADDING_IDX = 0

NC = 2
NS = 16
NW = NC * NS
PER_W = B // NW
K = PER_W * M
CH = 128
NCH = K // CH
L = 16
LB = 32
PK = 32

NBUF = 2
UNROLL = 4


def _sc_kernel(perts_hbm, dos_hbm, emb_hbm, beta_hbm, bias_hbm, out_hbm,
               idx_v, qidx_v, hidx_v, dos_v, betag_v, biasg_v, coeff_v,
               out_v, rows_v, sems):
    wid = lax.axis_index("s") * NC + lax.axis_index("c")
    base0 = wid * K

    pltpu.sync_copy(perts_hbm.at[pl.ds(base0, K)], idx_v.at[pl.ds(0, K)])
    pltpu.sync_copy(dos_hbm.at[pl.ds(base0, K)], dos_v)

    copies = []
    for j in range(NCH):
        idx_j = idx_v.at[pl.ds(j * CH, CH)]
        copies.append(pltpu.async_copy(
            beta_hbm.at[0].at[idx_j], betag_v.at[pl.ds(j * CH, CH)],
            sems.at[2]))
        copies.append(pltpu.async_copy(
            bias_hbm.at[0].at[idx_j], biasg_v.at[pl.ds(j * CH, CH)],
            sems.at[2]))
    for g in range(K // L):
        o = g * L
        p = idx_v[pl.ds(o, L)]
        qidx_v[pl.ds(o, L)] = p >> 2
        hidx_v[pl.ds(o, L)] = (p & 3) << 5
    for c in copies:
        c.wait()

    def fire(goff, buf):
        return pltpu.async_copy(
            emb_hbm.at[qidx_v.at[pl.ds(goff * CH, CH)]], rows_v.at[buf],
            sems.at[buf])

    def drain(buf):
        pltpu.make_async_copy(
            emb_hbm.at[qidx_v.at[pl.ds(0, CH)]], rows_v.at[buf],
            sems.at[buf]).wait()

    fire(0, 0)
    fire(1, 1)

    for g in range(K // L):
        o = g * L
        d = dos_v[pl.ds(o, L)]
        bg = betag_v[pl.ds(o, L)]
        hg = biasg_v[pl.ds(o, L)]
        p = idx_v[pl.ds(o, L)]
        t = d / (d + 2.0)
        t2 = t * t
        l1p = 2.0 * t * (1.0 + t2 * (1.0 / 3.0 + t2 * (
            0.2 + t2 * (1.0 / 7.0 + t2 * (1.0 / 9.0)))))
        z = l1p * bg + hg
        sg = 1.0 / (1.0 + jnp.exp(-z))
        s0 = 1.0 / (1.0 + jnp.exp(-hg))
        c = jnp.where(p == PADDING_IDX, 0.0, sg - s0)
        coeff_v[pl.ds(o, L)] = c

    hi_mask = jnp.full((L,), -65536, jnp.int32)

    def unpack_mul_add(b0, b1, c0, c1):
        l0 = plsc.bitcast(b0 << 16, jnp.float32)
        h0 = plsc.bitcast(b0 & hi_mask, jnp.float32)
        l1 = plsc.bitcast(b1 << 16, jnp.float32)
        h1 = plsc.bitcast(b1 & hi_mask, jnp.float32)
        lo = plsc.bitcast(c0 * l0 + c1 * l1, jnp.int32)
        hi = plsc.bitcast(c0 * h0 + c1 * h1, jnp.int32)
        return jax.lax.shift_right_logical(lo, 16) | (hi & hi_mask)

    def combine(g, buf):
        def pair_body(ii, _):
            for u in range(UNROLL):
                jj = ii * UNROLL + u
                k = g * CH + 2 * jj
                cv = coeff_v[pl.ds(k, L)]
                hv = hidx_v[pl.ds(k, L)]
                c0, c1 = cv[0], cv[1]
                h0, h1 = hv[0], hv[1]
                r0 = rows_v.at[buf].at[2 * jj]
                r1 = rows_v.at[buf].at[2 * jj + 1]
                o = out_v.at[lax.div(k, 2)]
                for q in range(2):
                    b0 = r0[pl.ds(h0 + q * L, L)]
                    b1 = r1[pl.ds(h1 + q * L, L)]
                    o[pl.ds(q * L, L)] = unpack_mul_add(b0, b1, c0, c1)
            return 0
        lax.fori_loop(0, CH // 2 // UNROLL, pair_body, 0)

    def outer(g2, _):
        g0 = g2 * NBUF
        for b in range(NBUF):
            g = g0 + b
            drain(b)
            combine(g, b)
            fire(jnp.minimum(g + NBUF, NCH - 1), b)
        return 0

    lax.fori_loop(0, NCH // NBUF, outer, 0)
    for b in range(NBUF):
        drain(b)

    pltpu.sync_copy(out_v, out_hbm.at[pl.ds(wid * PER_W, PER_W)])


@jax.jit
def kernel(perts, dosages, embedding, beta, bias):
    perts_f = perts.astype(jnp.int32).reshape(B * M)
    dos_f = dosages.astype(jnp.float32).reshape(B * M)
    emb_pk = lax.bitcast_convert_type(
        embedding.astype(jnp.bfloat16).reshape(N_PERTS // 4, 2 * N_LATENT,
                                               2),
        jnp.int32)

    mesh = plsc.VectorSubcoreMesh(core_axis_name="c", subcore_axis_name="s")
    fn = functools.partial(
        pl.kernel,
        mesh=mesh,
        compiler_params=pltpu.CompilerParams(needs_layout_passes=False),
        out_type=jax.ShapeDtypeStruct((B, PK), jnp.int32),
        scratch_types=[
            pltpu.VMEM((K + L,), jnp.int32),
            pltpu.VMEM((K,), jnp.int32),
            pltpu.VMEM((K + L,), jnp.int32),
            pltpu.VMEM((K,), jnp.float32),
            pltpu.VMEM((K,), jnp.float32),
            pltpu.VMEM((K,), jnp.float32),
            pltpu.VMEM((K + L,), jnp.float32),
            pltpu.VMEM((PER_W, PK), jnp.int32),
            pltpu.VMEM((NBUF, CH, 2 * N_LATENT), jnp.int32),
            pltpu.SemaphoreType.DMA((3,)),
        ],
    )(_sc_kernel)
    out_pk = fn(perts_f, dos_f, emb_pk, beta, bias)
    out_bf = lax.bitcast_convert_type(out_pk, jnp.bfloat16)
    return out_bf.reshape(B, N_LATENT).astype(jnp.float32)


# speedup vs baseline: 47.7053x; 47.7053x over previous
import functools

import jax
import jax.numpy as jnp
from jax import lax
from jax.experimental import pallas as pl
from jax.experimental.pallas import tpu as pltpu
from jax.experimental.pallas import tpu_sc as plsc

N_PERTS = 1000000
N_LATENT = 64
B = 16384
M = 2
PADDING_IDX = 0

NC = 2
NS = 16
NW = NC * NS
PER_W = B // NW
K = PER_W * M
CH = 128
NCH = K // CH
L = 16

NSLOT = 16
NROUND = PER_W // NSLOT - 1


def _sc_kernel(perts_hbm, dos_hbm, emb_hbm, beta_hbm, bias_hbm, out_hbm,
               idx_v, dos_v, betag_v, biasg_v, coeff_v, out_v, rows_v,
               gsem, sems):
    wid = lax.axis_index("s") * NC + lax.axis_index("c")
    base0 = wid * K

    pltpu.sync_copy(perts_hbm.at[pl.ds(base0, K)], idx_v.at[pl.ds(0, K)])
    pltpu.sync_copy(dos_hbm.at[pl.ds(base0, K)], dos_v)

    copies = []
    for j in range(NCH):
        idx_j = idx_v.at[pl.ds(j * CH, CH)]
        copies.append(pltpu.async_copy(
            beta_hbm.at[0].at[idx_j], betag_v.at[pl.ds(j * CH, CH)], gsem))
        copies.append(pltpu.async_copy(
            bias_hbm.at[0].at[idx_j], biasg_v.at[pl.ds(j * CH, CH)], gsem))
    for c in copies:
        c.wait()

    def fire(i, s):
        pv = idx_v[pl.ds(2 * i, L)]
        o0 = pl.multiple_of((pv[0] >> 3) * 8, 8)
        o1 = pl.multiple_of((pv[1] >> 3) * 8, 8)
        pltpu.async_copy(emb_hbm.at[pl.ds(o0, 8)], rows_v.at[s].at[0],
                         sems.at[s])
        pltpu.async_copy(emb_hbm.at[pl.ds(o1, 8)], rows_v.at[s].at[1],
                         sems.at[s])

    def drain(s):
        pltpu.make_async_copy(emb_hbm.at[pl.ds(0, 8)], rows_v.at[s].at[0],
                              sems.at[s]).wait()
        pltpu.make_async_copy(emb_hbm.at[pl.ds(0, 8)], rows_v.at[s].at[1],
                              sems.at[s]).wait()

    for s in range(NSLOT):
        fire(s, s)

    for g in range(K // L):
        o = g * L
        d = dos_v[pl.ds(o, L)]
        bg = betag_v[pl.ds(o, L)]
        hg = biasg_v[pl.ds(o, L)]
        p = idx_v[pl.ds(o, L)]
        t = d / (d + 2.0)
        t2 = t * t
        l1p = 2.0 * t * (1.0 + t2 * (1.0 / 3.0 + t2 * (
            0.2 + t2 * (1.0 / 7.0 + t2 * (1.0 / 9.0)))))
        z = l1p * bg + hg
        sg = 1.0 / (1.0 + jnp.exp(-z))
        s0 = 1.0 / (1.0 + jnp.exp(-hg))
        c = jnp.where(p == PADDING_IDX, 0.0, sg - s0)
        coeff_v[pl.ds(o, L)] = c

    def combine(i, s):
        cv = coeff_v[pl.ds(2 * i, L)]
        pv = idx_v[pl.ds(2 * i, L)]
        c0, c1 = cv[0], cv[1]
        r0 = rows_v.at[s].at[0].at[pv[0] & 7]
        r1 = rows_v.at[s].at[1].at[pv[1] & 7]
        o = out_v.at[i]
        for q in range(N_LATENT // L):
            sl = pl.ds(q * L, L)
            o[sl] = c0 * r0[sl] + c1 * r1[sl]

    def ring_round(c, _):
        base = c * NSLOT
        for s in range(NSLOT):
            i = base + s
            drain(s)
            combine(i, s)
            fire(i + NSLOT, s)
        return 0

    lax.fori_loop(0, NROUND, ring_round, 0)
    tail = NROUND * NSLOT
    for s in range(NSLOT):
        drain(s)
        combine(tail + s, s)

    pltpu.sync_copy(out_v, out_hbm.at[pl.ds(wid * PER_W, PER_W)])


@jax.jit
def kernel(perts, dosages, embedding, beta, bias):
    perts_f = perts.astype(jnp.int32).reshape(B * M)
    dos_f = dosages.astype(jnp.float32).reshape(B * M)

    mesh = plsc.VectorSubcoreMesh(core_axis_name="c", subcore_axis_name="s")
    fn = functools.partial(
        pl.kernel,
        mesh=mesh,
        out_type=jax.ShapeDtypeStruct((B, N_LATENT), jnp.float32),
        scratch_types=[
            pltpu.VMEM((K + L,), jnp.int32),
            pltpu.VMEM((K,), jnp.float32),
            pltpu.VMEM((K,), jnp.float32),
            pltpu.VMEM((K,), jnp.float32),
            pltpu.VMEM((K + L,), jnp.float32),
            pltpu.VMEM((PER_W, N_LATENT), jnp.float32),
            pltpu.VMEM((NSLOT, M, 8, N_LATENT), jnp.float32),
            pltpu.SemaphoreType.DMA,
            pltpu.SemaphoreType.DMA((NSLOT,)),
        ],
    )(_sc_kernel)
    return fn(perts_f, dos_f, embedding, beta, bias)


# speedup vs baseline: 48.6820x; 1.0205x over previous
import functools

import jax
import jax.numpy as jnp
from jax import lax
from jax.experimental import pallas as pl
from jax.experimental.pallas import tpu as pltpu
from jax.experimental.pallas import tpu_sc as plsc

N_PERTS = 1000000
N_LATENT = 64
B = 16384
M = 2
PADDING_IDX = 0

NC = 2
NS = 16
NW = NC * NS
PER_W = B // NW
K = PER_W * M
CH = 128
NCH = K // CH
L = 16
LB = 32

NSLOT = 32
NROUND = PER_W // NSLOT - 1
assert PER_W % NSLOT == 0


def _sc_kernel(perts_hbm, dos_hbm, emb_hbm, beta_hbm, bias_hbm, out_hbm,
               idx_v, dos_v, betag_v, biasg_v, coeff_v, out_v, rows_v,
               gsem, sems):
    wid = lax.axis_index("s") * NC + lax.axis_index("c")
    base0 = wid * K

    pltpu.sync_copy(perts_hbm.at[pl.ds(base0, K)], idx_v.at[pl.ds(0, K)])
    pltpu.sync_copy(dos_hbm.at[pl.ds(base0, K)], dos_v)

    copies = []
    for j in range(NCH):
        idx_j = idx_v.at[pl.ds(j * CH, CH)]
        copies.append(pltpu.async_copy(
            beta_hbm.at[0].at[idx_j], betag_v.at[pl.ds(j * CH, CH)], gsem))
        copies.append(pltpu.async_copy(
            bias_hbm.at[0].at[idx_j], biasg_v.at[pl.ds(j * CH, CH)], gsem))
    for c in copies:
        c.wait()

    def fire(i, s):
        pv = idx_v[pl.ds(2 * i, L)]
        o0 = pl.multiple_of((pv[0] >> 3) * 8, 8)
        o1 = pl.multiple_of((pv[1] >> 3) * 8, 8)
        pltpu.async_copy(emb_hbm.at[pl.ds(o0, 8)], rows_v.at[s].at[0],
                         sems.at[s // 2])
        pltpu.async_copy(emb_hbm.at[pl.ds(o1, 8)], rows_v.at[s].at[1],
                         sems.at[s // 2])

    def drain(s):
        pltpu.make_async_copy(emb_hbm.at[pl.ds(0, 8)], rows_v.at[s].at[0],
                              sems.at[s // 2]).wait()
        pltpu.make_async_copy(emb_hbm.at[pl.ds(0, 8)], rows_v.at[s].at[1],
                              sems.at[s // 2]).wait()

    for s in range(NSLOT):
        fire(s, s)

    for g in range(K // L):
        o = g * L
        d = dos_v[pl.ds(o, L)]
        bg = betag_v[pl.ds(o, L)]
        hg = biasg_v[pl.ds(o, L)]
        p = idx_v[pl.ds(o, L)]
        t = d / (d + 2.0)
        t2 = t * t
        l1p = 2.0 * t * (1.0 + t2 * (1.0 / 3.0 + t2 * (
            0.2 + t2 * (1.0 / 7.0 + t2 * (1.0 / 9.0)))))
        z = l1p * bg + hg
        sg = 1.0 / (1.0 + jnp.exp(-z))
        s0 = 1.0 / (1.0 + jnp.exp(-hg))
        c = jnp.where(p == PADDING_IDX, 0.0, sg - s0)
        coeff_v[pl.ds(o, L)] = c

    def combine(i, s):
        cv = coeff_v[pl.ds(2 * i, L)]
        pv = idx_v[pl.ds(2 * i, L)]
        c0, c1 = cv[0], cv[1]
        r0 = rows_v.at[s].at[0].at[pv[0] & 7]
        r1 = rows_v.at[s].at[1].at[pv[1] & 7]
        ob = i * N_LATENT
        for q in range(N_LATENT // L):
            sl = pl.ds(q * L, L)
            out_v[pl.ds(ob + q * L, L)] = c0 * r0[sl] + c1 * r1[sl]

    def ring_round(c, _):
        base = c * NSLOT
        for sp in range(NSLOT // 2):
            s0, s1 = 2 * sp, 2 * sp + 1
            drain(s0)
            drain(s1)
            combine(base + s0, s0)
            combine(base + s1, s1)
            fire(base + s0 + NSLOT, s0)
            fire(base + s1 + NSLOT, s1)
        return 0

    lax.fori_loop(0, NROUND, ring_round, 0)
    tail = NROUND * NSLOT
    for s in range(NSLOT):
        drain(s)
        combine(tail + s, s)

    pltpu.sync_copy(
        out_v,
        out_hbm.at[pl.ds(pl.multiple_of(wid * (PER_W * N_LATENT),
                                        PER_W * N_LATENT),
                         PER_W * N_LATENT)])


@jax.jit
def kernel(perts, dosages, embedding, beta, bias):
    perts_f = perts.astype(jnp.int32).reshape(B * M)
    dos_f = dosages.astype(jnp.float32).reshape(B * M)

    mesh = plsc.VectorSubcoreMesh(core_axis_name="c", subcore_axis_name="s")
    fn = functools.partial(
        pl.kernel,
        mesh=mesh,
        out_type=jax.ShapeDtypeStruct((B * N_LATENT,), jnp.float32),
        scratch_types=[
            pltpu.VMEM((K + L,), jnp.int32),
            pltpu.VMEM((K,), jnp.float32),
            pltpu.VMEM((K,), jnp.float32),
            pltpu.VMEM((K,), jnp.float32),
            pltpu.VMEM((K + L,), jnp.float32),
            pltpu.VMEM((PER_W * N_LATENT,), jnp.float32),
            pltpu.VMEM((NSLOT, M, 8, N_LATENT), jnp.float32),
            pltpu.SemaphoreType.DMA,
            pltpu.SemaphoreType.DMA((NSLOT // 2,)),
        ],
    )(_sc_kernel)
    return fn(perts_f, dos_f, embedding, beta, bias).reshape(B, N_LATENT)
